# Initial kernel scaffold; baseline (speedup 1.0000x reference)
import jax
import jax.numpy as jnp
from jax.experimental import pallas as pl


def kernel(x, We1, be1, We2, be2, We3, be3, E, Wd1, bd1, Wd2, bd2, Wd3, bd3):
    raise NotImplementedError("write your pallas kernel here")


# speedup vs baseline: 1.2457x; 1.2457x over previous
import functools

import jax
import jax.numpy as jnp
from jax import lax
from jax.experimental import pallas as pl
from jax.experimental.pallas import tpu as pltpu
from jax.experimental.pallas import tpu_sc as plsc

_B = 16384
_IN = 512
_HID = 1024
_MID = 128
_LAT = 256
_K = 8192
_CC = 0.25

_BM1 = 256
_NB1 = _B // _BM1
_BM3 = 512
_NB3 = _B // _BM3

_NC = 2
_NS = 16
_NW = _NC * _NS
_BPW = _B // _NW
_CH = 128
_NCH = _BPW // _CH


def _mm(a, b, dims):
    return lax.dot_general(a, b, (dims, ((), ())),
                           preferred_element_type=jnp.float32)


def _enc_body(x_ref, we1_ref, be1_ref, we2_ref, be2_ref, we3_ref, be3_ref,
              e_ref, e2_ref, ze_ref, idx_ref):
    h = jnp.maximum(_mm(x_ref[...], we1_ref[...], ((1,), (0,))) + be1_ref[...], 0.0)
    h = jnp.maximum(_mm(h, we2_ref[...], ((1,), (0,))) + be2_ref[...], 0.0)
    z = _mm(h, we3_ref[...], ((1,), (0,))) + be3_ref[...]
    ze_ref[...] = z
    s = _mm(z, e_ref[...], ((1,), (1,)))
    d = e2_ref[...] - 2.0 * s
    m = jnp.min(d, axis=1, keepdims=True)
    ids = lax.broadcasted_iota(jnp.int32, (_BM1, _K), 1)
    sel = jnp.where(d == m, ids, _K)
    idx_ref[...] = jnp.min(sel, axis=1).reshape(1, 1, _BM1)


def _run_stage1(x, We1, be1, We2, be2, We3, be3, E, e2, interpret=False):
    full = lambda shape: pl.BlockSpec(shape, lambda i: (0,) * len(shape))
    return pl.pallas_call(
        _enc_body,
        grid=(_NB1,),
        in_specs=[
            pl.BlockSpec((_BM1, _IN), lambda i: (i, 0)),
            full((_IN, _HID)), full((1, _HID)),
            full((_HID, _MID)), full((1, _MID)),
            full((_MID, _LAT)), full((1, _LAT)),
            full((_K, _LAT)), full((1, _K)),
        ],
        out_specs=[
            pl.BlockSpec((_BM1, _LAT), lambda i: (i, 0)),
            pl.BlockSpec((1, 1, _BM1), lambda i: (i, 0, 0)),
        ],
        out_shape=[
            jax.ShapeDtypeStruct((_B, _LAT), jnp.float32),
            jax.ShapeDtypeStruct((_NB1, 1, _BM1), jnp.int32),
        ],
        compiler_params=pltpu.CompilerParams(
            dimension_semantics=("arbitrary",)),
        interpret=interpret,
    )(x, We1, be1, We2, be2, We3, be3, E, e2)


def _sc_body(e_hbm, idx_hbm, z0_hbm, q_hbm, cnt_hbm,
             idx_v, rows_v, ones_v, shared, sem):
    c = lax.axis_index("c")
    s = lax.axis_index("s")
    wid = s * _NC + c
    base = wid * _BPW
    for j in range(_NCH):
        pltpu.sync_copy(idx_hbm.at[pl.ds(base + j * _CH, _CH)], idx_v.at[j])
    for j in range(_NCH):
        pltpu.async_copy(e_hbm.at[idx_v.at[j]], rows_v, sem).wait()
        pltpu.sync_copy(rows_v, q_hbm.at[pl.ds(base + j * _CH, _CH)])
    for t in range(_CH // 16):
        ones_v[pl.ds(t * 16, 16)] = jnp.full((16,), 1.0, jnp.float32)

    @pl.when(s == 0)
    def _():
        pltpu.sync_copy(z0_hbm.at[c], shared)

    plsc.subcore_barrier()
    for j in range(_NCH):
        pltpu.sync_copy(ones_v, shared.at[idx_v.at[j]], add=True)
    plsc.subcore_barrier()

    @pl.when(s == 0)
    def _():
        pltpu.sync_copy(shared, cnt_hbm.at[c])


@functools.cache
def _sc_gather_counts():
    return pl.kernel(
        _sc_body,
        out_type=[
            jax.ShapeDtypeStruct((_B, _LAT), jnp.float32),
            jax.ShapeDtypeStruct((_NC, _K), jnp.float32),
        ],
        mesh=plsc.VectorSubcoreMesh(core_axis_name="c", subcore_axis_name="s"),
        scratch_types=[
            pltpu.VMEM((_NCH, _CH), jnp.int32),
            pltpu.VMEM((_CH, _LAT), jnp.float32),
            pltpu.VMEM((_CH,), jnp.float32),
            pltpu.VMEM_SHARED((_K,), jnp.float32),
            pltpu.SemaphoreType.DMA,
        ],
    )


def _dec_body(q_ref, z_ref, wd1_ref, bd1_ref, wd2_ref, bd2_ref, wd3_ref,
              bd3_ref, cnt_ref, xr_ref, vq_ref, pp_ref):
    i = pl.program_id(0)

    @pl.when(i == 0)
    def _():
        vq_ref[...] = jnp.zeros((1, 1), jnp.float32)

    q = q_ref[...]
    diff = q - z_ref[...]
    vq_ref[...] += jnp.sum(diff * diff, keepdims=True)
    d = jnp.maximum(_mm(q, wd1_ref[...], ((1,), (0,))) + bd1_ref[...], 0.0)
    d = jnp.maximum(_mm(d, wd2_ref[...], ((1,), (0,))) + bd2_ref[...], 0.0)
    xr_ref[...] = _mm(d, wd3_ref[...], ((1,), (0,))) + bd3_ref[...]

    @pl.when(i == _NB3 - 1)
    def _():
        vq_ref[...] = vq_ref[...] * ((1.0 + _CC) / (_B * _LAT))
        p = (cnt_ref[0:1, :] + cnt_ref[1:2, :]) * (1.0 / _B)
        ent = -jnp.sum(p * jnp.log(p + 1e-10), keepdims=True)
        pp_ref[...] = jnp.exp(ent)


def _run_stage3(quant, z_e, Wd1, bd1, Wd2, bd2, Wd3, bd3, counts,
                interpret=False):
    full = lambda shape: pl.BlockSpec(shape, lambda i: (0,) * len(shape))
    return pl.pallas_call(
        _dec_body,
        grid=(_NB3,),
        in_specs=[
            pl.BlockSpec((_BM3, _LAT), lambda i: (i, 0)),
            pl.BlockSpec((_BM3, _LAT), lambda i: (i, 0)),
            full((_LAT, _MID)), full((1, _MID)),
            full((_MID, _HID)), full((1, _HID)),
            full((_HID, _IN)), full((1, _IN)),
            full((_NC, _K)),
        ],
        out_specs=[
            pl.BlockSpec((_BM3, _IN), lambda i: (i, 0)),
            full((1, 1)),
            full((1, 1)),
        ],
        out_shape=[
            jax.ShapeDtypeStruct((_B, _IN), jnp.float32),
            jax.ShapeDtypeStruct((1, 1), jnp.float32),
            jax.ShapeDtypeStruct((1, 1), jnp.float32),
        ],
        compiler_params=pltpu.CompilerParams(
            dimension_semantics=("arbitrary",)),
        interpret=interpret,
    )(quant, z_e, Wd1, bd1, Wd2, bd2, Wd3, bd3, counts)


def kernel(x, We1, be1, We2, be2, We3, be3, E, Wd1, bd1, Wd2, bd2, Wd3, bd3):
    e2 = jnp.sum(E ** 2, axis=1).reshape(1, _K)
    z_e, idx3 = _run_stage1(x, We1, be1.reshape(1, _HID), We2,
                            be2.reshape(1, _MID), We3, be3.reshape(1, _LAT),
                            E, e2)
    idx = idx3.reshape(_B)
    zeros2 = jnp.zeros((_NC, _K), jnp.float32)
    quant, counts = _sc_gather_counts()(E, idx, zeros2)
    x_recon, vq, pp = _run_stage3(quant, z_e, Wd1, bd1.reshape(1, _MID), Wd2,
                                  bd2.reshape(1, _HID), Wd3,
                                  bd3.reshape(1, _IN), counts)
    return x_recon, vq[0, 0], pp[0, 0]
